# baseline (speedup 1.0000x reference)
import jax
import jax.numpy as jnp
from jax.experimental import pallas as pl

_NUM_NODES = 1000000
_DIM = 64
_BLOCK_ROWS = 8000


def _copy_block(x_ref, o_ref):
    o_ref[...] = x_ref[...]


def kernel(emb_weight):
    grid = (_NUM_NODES // _BLOCK_ROWS,)
    return pl.pallas_call(
        _copy_block,
        out_shape=jax.ShapeDtypeStruct((_NUM_NODES, _DIM), jnp.float32),
        grid=grid,
        in_specs=[pl.BlockSpec((_BLOCK_ROWS, _DIM), lambda i: (i, 0))],
        out_specs=pl.BlockSpec((_BLOCK_ROWS, _DIM), lambda i: (i, 0)),
    )(emb_weight)
